# Initial kernel scaffold; baseline (speedup 1.0000x reference)
import jax
import jax.numpy as jnp
from jax.experimental import pallas as pl


def kernel(x, edge_index, W1_l, b1, W1_r, W2_l, b2, W2_r):
    raise NotImplementedError("write your pallas kernel here")


# speedup vs baseline: 9.0775x; 9.0775x over previous
import functools

import jax
import jax.numpy as jnp
from jax import lax
from jax.experimental import pallas as pl
from jax.experimental.pallas import tpu as pltpu
from jax.experimental.pallas import tpu_sc as plsc

N_NODES = 10000
N_EDGES = 320000
IN_DIM = 128
HID_DIM = 64
OUT_DIM = 3

W1P = 80
W2P = 16

NC = 2
NS = 16
NW = NC * NS
CHUNK = 128
NCH = N_EDGES // CHUNK
STRIPE = 624
TAIL = N_NODES - NS * STRIPE


def _lin1_body(x_ref, wl_ref, wr_ref, b_ref, y1p_ref, xr1_ref):
    x = x_ref[...]
    y = lax.dot_general(x, wl_ref[...], (((1,), (1,)), ((), ())),
                        preferred_element_type=jnp.float32)
    n = x.shape[0]
    y1p_ref[...] = jnp.concatenate(
        [y, jnp.ones((n, 1), jnp.float32),
         jnp.zeros((n, W1P - HID_DIM - 1), jnp.float32)], axis=1)
    xr1_ref[...] = lax.dot_general(x, wr_ref[...], (((1,), (1,)), ((), ())),
                                   preferred_element_type=jnp.float32) + b_ref[...]


def _lin1(x, wl, wr, b):
    return pl.pallas_call(
        _lin1_body,
        out_shape=(
            jax.ShapeDtypeStruct((N_NODES, W1P), jnp.float32),
            jax.ShapeDtypeStruct((N_NODES, HID_DIM), jnp.float32),
        ),
    )(x, wl, wr, b)


def _lin2_body(aggp_ref, xr1_ref, wl_ref, wr_ref, b_ref,
               y2p_ref, r2_ref, dinv_ref):
    agg = aggp_ref[0] + aggp_ref[1]
    deg = agg[:, HID_DIM:HID_DIM + 1]
    dinv = 1.0 / jnp.maximum(deg, 1.0)
    h = jnp.maximum(agg[:, :HID_DIM] * dinv + xr1_ref[...], 0.0)
    y2 = lax.dot_general(h, wl_ref[...], (((1,), (1,)), ((), ())),
                         preferred_element_type=jnp.float32)
    n = h.shape[0]
    y2p_ref[...] = jnp.concatenate(
        [y2, jnp.zeros((n, W2P - OUT_DIM), jnp.float32)], axis=1)
    r2_ref[...] = lax.dot_general(h, wr_ref[...], (((1,), (1,)), ((), ())),
                                  preferred_element_type=jnp.float32) + b_ref[...]
    dinv_ref[...] = dinv


def _lin2(aggp, xr1, wl, wr, b):
    return pl.pallas_call(
        _lin2_body,
        out_shape=(
            jax.ShapeDtypeStruct((N_NODES, W2P), jnp.float32),
            jax.ShapeDtypeStruct((N_NODES, OUT_DIM), jnp.float32),
            jax.ShapeDtypeStruct((N_NODES, 1), jnp.float32),
        ),
    )(aggp, xr1, wl, wr, b)


def _final_body(aggp_ref, r2_ref, dinv_ref, o_ref):
    agg = aggp_ref[0] + aggp_ref[1]
    z = agg[:, :OUT_DIM] * dinv_ref[...] + r2_ref[...]
    m = jnp.max(z, axis=1, keepdims=True)
    lse = jnp.log(jnp.sum(jnp.exp(z - m), axis=1, keepdims=True)) + m
    o_ref[...] = z - lse


def _final(aggp, r2, dinv):
    return pl.pallas_call(
        _final_body,
        out_shape=jax.ShapeDtypeStruct((N_NODES, OUT_DIM), jnp.float32),
    )(aggp, r2, dinv)


def _make_scatter(width):
    mesh = plsc.VectorSubcoreMesh(core_axis_name="c", subcore_axis_name="s")

    @functools.partial(
        pl.kernel,
        mesh=mesh,
        compiler_params=pltpu.CompilerParams(use_tc_tiling_on_sc=False),
        out_type=jax.ShapeDtypeStruct((NC, N_NODES, width), jnp.float32),
        scratch_types=[
            pltpu.VMEM((2, CHUNK), jnp.int32),
            pltpu.VMEM((CHUNK, width), jnp.float32),
            pltpu.VMEM_SHARED((N_NODES, width), jnp.float32),
            pltpu.SemaphoreType.DMA,
        ],
    )
    def scat(edges_hbm, tab_hbm, zeros_hbm, out_hbm, idx_v, rows_v, acc_sh, sem):
        c = lax.axis_index("c")
        s = lax.axis_index("s")
        w = s * NC + c
        r0 = s * STRIPE
        pltpu.sync_copy(zeros_hbm.at[pl.ds(r0, STRIPE)],
                        acc_sh.at[pl.ds(r0, STRIPE)])

        @pl.when(s == NS - 1)
        def _():
            pltpu.sync_copy(zeros_hbm.at[pl.ds(NS * STRIPE, TAIL)],
                            acc_sh.at[pl.ds(NS * STRIPE, TAIL)])

        plsc.subcore_barrier()

        nch = jnp.where(w < (NCH % NW), NCH // NW + 1, NCH // NW)

        def body(j, _):
            off = (j * NW + w) * CHUNK
            pltpu.sync_copy(edges_hbm.at[:, pl.ds(off, CHUNK)], idx_v)
            pltpu.async_copy(tab_hbm.at[idx_v.at[0]], rows_v, sem).wait()
            pltpu.sync_copy(rows_v, acc_sh.at[idx_v.at[1]], add=True)
            return 0

        lax.fori_loop(0, nch, body, 0)
        plsc.subcore_barrier()
        pltpu.sync_copy(acc_sh.at[pl.ds(r0, STRIPE)],
                        out_hbm.at[c, pl.ds(r0, STRIPE)])

        @pl.when(s == NS - 1)
        def _():
            pltpu.sync_copy(acc_sh.at[pl.ds(NS * STRIPE, TAIL)],
                            out_hbm.at[c, pl.ds(NS * STRIPE, TAIL)])

    return scat


_scatter1 = _make_scatter(W1P)
_scatter2 = _make_scatter(W2P)


def kernel(x, edge_index, W1_l, b1, W1_r, W2_l, b2, W2_r):
    zeros1 = jnp.zeros((N_NODES, W1P), jnp.float32)
    zeros2 = jnp.zeros((N_NODES, W2P), jnp.float32)
    y1p, xr1 = _lin1(x, W1_l, W1_r, b1.reshape(1, HID_DIM))
    agg1p = _scatter1(edge_index, y1p, zeros1)
    y2p, r2, dinv = _lin2(agg1p, xr1, W2_l, W2_r, b2.reshape(1, OUT_DIM))
    agg2p = _scatter2(edge_index, y2p, zeros2)
    return _final(agg2p, r2, dinv)
